# baseline (speedup 1.0000x reference)
import functools

import jax
import jax.numpy as jnp
from jax import lax
from jax.experimental import pallas as pl
from jax.experimental.pallas import tpu as pltpu
from jax.experimental.pallas import tpu_sc as plsc

_N = 10000
_D = 128
_E = 320000
_NC = 2
_NS = 16
_NW = _NC * _NS
_EW = _E // _NW
_K = 80
_NCHUNK = _EW // _K
_NPAD = 10240
_RPT = _NPAD // _NS
_FB = 32
_BR = 2000
_G = _N // _BR


def _seg_body(with_deg, *refs):
    if with_deg:
        (h_hbm, src_hbm, dst_hbm, out0, out1, deg0, deg1,
         src_all, dstv0, dstv1, dstv2, rows0, rows1, rows2, fb, acc,
         ones_v, degfb, dacc,
         semI0, semI1, semI2, semG0, semG1, semG2) = refs
    else:
        (h_hbm, src_hbm, dst_hbm, out0, out1,
         src_all, dstv0, dstv1, dstv2, rows0, rows1, rows2, fb, acc,
         semI0, semI1, semI2, semG0, semG1, semG2) = refs
        ones_v = degfb = dacc = deg0 = deg1 = None
    bufs = ((dstv0, rows0, semI0, semG0),
            (dstv1, rows1, semI1, semG1),
            (dstv2, rows2, semI2, semG2))

    c = lax.axis_index("c")
    s = lax.axis_index("s")
    wid = s * _NC + c
    r0 = s * _RPT

    z16 = jnp.zeros((16,), jnp.float32)

    def _zrow(i, carry):
        for j in range(_D // 16):
            fb[i, pl.ds(j * 16, 16)] = z16
        return carry

    lax.fori_loop(0, _FB, _zrow, 0)
    for t in range(_RPT // _FB):
        pltpu.sync_copy(fb, acc.at[pl.ds(r0 + t * _FB, _FB)])

    if with_deg:
        def _zdeg(i, carry):
            degfb[pl.ds(i * 16, 16)] = z16
            return carry

        lax.fori_loop(0, _RPT // 16, _zdeg, 0)
        pltpu.sync_copy(degfb, dacc.at[pl.ds(r0, _RPT)])
        o16 = jnp.ones((16,), jnp.float32)
        for j in range(_K // 16):
            ones_v[pl.ds(j * 16, 16)] = o16

    pltpu.sync_copy(src_hbm.at[pl.ds(wid * _EW, _EW)], src_all)

    plsc.subcore_barrier()

    def _issue(j, b):
        dstv, rows, semI, semG = bufs[b]
        pltpu.async_copy(dst_hbm.at[pl.ds(pl.multiple_of(wid * _EW + j * _K, 8), _K)], dstv, semI)
        idx = src_all.at[pl.ds(pl.multiple_of(j * _K, 8), _K)]
        pltpu.async_copy(h_hbm.at[idx], rows, semG)

    def _drain_scatter(b):
        dstv, rows, semI, semG = bufs[b]
        pltpu.make_async_copy(dst_hbm.at[pl.ds(0, _K)], dstv, semI).wait()
        idx = src_all.at[pl.ds(0, _K)]
        pltpu.make_async_copy(h_hbm.at[idx], rows, semG).wait()
        pltpu.sync_copy(rows, acc.at[dstv], add=True)
        if with_deg:
            pltpu.sync_copy(ones_v, dacc.at[dstv], add=True)

    _issue(0, 0)
    _issue(1, 1)
    _issue(2, 2)

    def _triple(t, carry):
        j0 = 3 * t
        for u in range(3):
            _drain_scatter(u)

            @pl.when(j0 + u + 3 < _NCHUNK)
            def _():
                _issue(j0 + u + 3, u)
        return carry

    lax.fori_loop(0, _NCHUNK // 3, _triple, 0)
    for u in range(_NCHUNK % 3):
        _drain_scatter(u)

    plsc.subcore_barrier()

    def _flush(dst_out):
        pltpu.sync_copy(acc.at[pl.ds(r0, _RPT)], dst_out.at[pl.ds(r0, _RPT)])

    @pl.when(c == 0)
    def _():
        _flush(out0)

    @pl.when(c == 1)
    def _():
        _flush(out1)

    if with_deg:
        @pl.when(c == 0)
        def _():
            pltpu.sync_copy(dacc.at[pl.ds(r0, _RPT)], deg0.at[pl.ds(r0, _RPT)])

        @pl.when(c == 1)
        def _():
            pltpu.sync_copy(dacc.at[pl.ds(r0, _RPT)], deg1.at[pl.ds(r0, _RPT)])


def _make_seg(with_deg):
    mesh = plsc.VectorSubcoreMesh(core_axis_name="c", subcore_axis_name="s")
    out_type = [jax.ShapeDtypeStruct((_NPAD, _D), jnp.float32)] * 2
    scratch = [
        pltpu.VMEM((_EW,), jnp.int32),
        pltpu.VMEM((_K,), jnp.int32),
        pltpu.VMEM((_K,), jnp.int32),
        pltpu.VMEM((_K,), jnp.int32),
        pltpu.VMEM((_K, _D), jnp.float32),
        pltpu.VMEM((_K, _D), jnp.float32),
        pltpu.VMEM((_K, _D), jnp.float32),
        pltpu.VMEM((_FB, _D), jnp.float32),
        pltpu.VMEM_SHARED((_NPAD, _D), jnp.float32),
    ]
    if with_deg:
        out_type += [jax.ShapeDtypeStruct((_NPAD,), jnp.float32)] * 2
        scratch += [
            pltpu.VMEM((_K,), jnp.float32),
            pltpu.VMEM((_RPT,), jnp.float32),
            pltpu.VMEM_SHARED((_NPAD,), jnp.float32),
        ]
    scratch += [pltpu.SemaphoreType.DMA] * 6
    return pl.kernel(
        functools.partial(_seg_body, with_deg),
        out_type=tuple(out_type),
        mesh=mesh,
        scratch_types=scratch,
    )


_seg_with_deg = _make_seg(True)
_seg_no_deg = _make_seg(False)


def _enc_body(x, w, b, ei, o, so, do_):
    o[...] = jnp.maximum(
        jnp.dot(x[...], w[...], preferred_element_type=jnp.float32) + b[...], 0.0)
    so[...] = ei[0]
    do_[...] = ei[1]


def _layer_body(p0, p1, d0, d1, w, b, o):
    rdeg = 1.0 / jnp.maximum(d0[pl.ds(0, _N), :] + d1[pl.ds(0, _N), :], 1.0)
    agg = (p0[pl.ds(0, _N), :] + p1[pl.ds(0, _N), :]) * rdeg
    hh = jnp.concatenate(
        [jnp.dot(agg, w[0], preferred_element_type=jnp.float32),
         jnp.dot(agg, w[1], preferred_element_type=jnp.float32)], axis=-1)
    o[...] = jnp.maximum(hh + b[...], 0.0)


def _final_body(p0, p1, d0, d1, w, b, wd, bd, out_o, h_o):
    rdeg = 1.0 / jnp.maximum(d0[pl.ds(0, _N), :] + d1[pl.ds(0, _N), :], 1.0)
    agg = (p0[pl.ds(0, _N), :] + p1[pl.ds(0, _N), :]) * rdeg
    hh = jnp.concatenate(
        [jnp.dot(agg, w[0], preferred_element_type=jnp.float32),
         jnp.dot(agg, w[1], preferred_element_type=jnp.float32)], axis=-1)
    hh = jnp.maximum(hh + b[...], 0.0)
    h_o[...] = hh
    out_o[...] = jnp.dot(hh, wd[...], preferred_element_type=jnp.float32) + bd[...]


_row_spec = pl.BlockSpec((_BR, _D), lambda i: (i, 0))
_deg_spec = pl.BlockSpec((_BR, 1), lambda i: (i, 0))
_w_spec = pl.BlockSpec((_D, _D), lambda i: (0, 0))
_wh_spec = pl.BlockSpec((2, _D, _D // 2), lambda i: (0, 0, 0))
_b_spec = pl.BlockSpec((1, _D), lambda i: (0, 0))

_enc_call = pl.pallas_call(
    _enc_body,
    out_shape=(jax.ShapeDtypeStruct((_N, _D), jnp.float32),
               jax.ShapeDtypeStruct((_E,), jnp.int32),
               jax.ShapeDtypeStruct((_E,), jnp.int32)),
)

_layer_call = pl.pallas_call(
    _layer_body,
    out_shape=jax.ShapeDtypeStruct((_N, _D), jnp.float32),
)

_final_call = pl.pallas_call(
    _final_body,
    out_shape=(jax.ShapeDtypeStruct((_N, _D), jnp.float32),
               jax.ShapeDtypeStruct((_N, _D), jnp.float32)),
)


def kernel(x, edge_index, enc_W, enc_b, conv_Ws, conv_bs, dec_W, dec_b):


    h0, src1, dst1 = _enc_call(x, enc_W, enc_b.reshape(1, _D), edge_index)

    pa, pb, dg0, dg1 = _seg_with_deg(h0, src1, dst1)
    d0 = dg0.reshape(_NPAD, 1)
    d1 = dg1.reshape(_NPAD, 1)

    h1 = _layer_call(pa, pb, d0, d1, conv_Ws[0], conv_bs[0].reshape(1, _D))

    qa, qb = _seg_no_deg(h1, src1, dst1)

    out, h2 = _final_call(qa, qb, d0, d1, conv_Ws[1], conv_bs[1].reshape(1, _D),
                          dec_W, dec_b.reshape(1, _D))
    return (out, h2)


# speedup vs baseline: 1.0079x; 1.0079x over previous
import functools

import jax
import jax.numpy as jnp
from jax import lax
from jax.experimental import pallas as pl
from jax.experimental.pallas import tpu as pltpu
from jax.experimental.pallas import tpu_sc as plsc

_N = 10000
_D = 128
_E = 320000
_NC = 2
_NS = 16
_NW = _NC * _NS
_EW = _E // _NW
_K = 80
_NCHUNK = _EW // _K
_NPAD = 10240
_RPT = _NPAD // _NS
_FB = 32
_BR = 2000
_G = _N // _BR


def _seg_body(with_deg, *refs):
    if with_deg:
        (h_hbm, src_hbm, dst_hbm, out0, out1, deg0, deg1,
         src_all, dstv0, dstv1, dstv2, rows0, rows1, rows2, fb, acc,
         ones_v, degfb, dacc,
         semI0, semI1, semI2, semG0, semG1, semG2) = refs
    else:
        (h_hbm, src_hbm, dst_hbm, out0, out1,
         src_all, dstv0, dstv1, dstv2, rows0, rows1, rows2, fb, acc,
         semI0, semI1, semI2, semG0, semG1, semG2) = refs
        ones_v = degfb = dacc = deg0 = deg1 = None
    bufs = ((dstv0, rows0, semI0, semG0),
            (dstv1, rows1, semI1, semG1),
            (dstv2, rows2, semI2, semG2))

    c = lax.axis_index("c")
    s = lax.axis_index("s")
    wid = s * _NC + c
    r0 = s * _RPT

    z16 = jnp.zeros((16,), jnp.float32)

    def _zrow(i, carry):
        for j in range(_D // 16):
            fb[i, pl.ds(j * 16, 16)] = z16
        return carry

    lax.fori_loop(0, _FB, _zrow, 0)
    for t in range(_RPT // _FB):
        pltpu.sync_copy(fb, acc.at[pl.ds(r0 + t * _FB, _FB)])

    if with_deg:
        def _zdeg(i, carry):
            degfb[pl.ds(i * 16, 16)] = z16
            return carry

        lax.fori_loop(0, _RPT // 16, _zdeg, 0)
        pltpu.sync_copy(degfb, dacc.at[pl.ds(r0, _RPT)])
        o16 = jnp.ones((16,), jnp.float32)
        for j in range(_K // 16):
            ones_v[pl.ds(j * 16, 16)] = o16

    pltpu.sync_copy(src_hbm.at[pl.ds(wid * _EW, _EW)], src_all)

    plsc.subcore_barrier()

    def _issue(j, b):
        dstv, rows, semI, semG = bufs[b]
        pltpu.async_copy(dst_hbm.at[pl.ds(pl.multiple_of(wid * _EW + j * _K, 8), _K)], dstv, semI)
        idx = src_all.at[pl.ds(pl.multiple_of(j * _K, 8), _K)]
        pltpu.async_copy(h_hbm.at[idx], rows, semG)

    def _drain_scatter(b):
        dstv, rows, semI, semG = bufs[b]
        pltpu.make_async_copy(dst_hbm.at[pl.ds(0, _K)], dstv, semI).wait()
        idx = src_all.at[pl.ds(0, _K)]
        pltpu.make_async_copy(h_hbm.at[idx], rows, semG).wait()
        pltpu.sync_copy(rows, acc.at[dstv], add=True)
        if with_deg:
            pltpu.sync_copy(ones_v, dacc.at[dstv], add=True)

    _issue(0, 0)
    _issue(1, 1)
    _issue(2, 2)

    def _triple(t, carry):
        j0 = 3 * t
        for u in range(3):
            _drain_scatter(u)

            @pl.when(j0 + u + 3 < _NCHUNK)
            def _():
                _issue(j0 + u + 3, u)
        return carry

    lax.fori_loop(0, _NCHUNK // 3, _triple, 0)
    for u in range(_NCHUNK % 3):
        _drain_scatter(u)

    plsc.subcore_barrier()

    def _flush(dst_out):
        pltpu.sync_copy(acc.at[pl.ds(r0, _RPT)], dst_out.at[pl.ds(r0, _RPT)])

    @pl.when(c == 0)
    def _():
        _flush(out0)

    @pl.when(c == 1)
    def _():
        _flush(out1)

    if with_deg:
        @pl.when(c == 0)
        def _():
            pltpu.sync_copy(dacc.at[pl.ds(r0, _RPT)], deg0.at[pl.ds(r0, _RPT)])

        @pl.when(c == 1)
        def _():
            pltpu.sync_copy(dacc.at[pl.ds(r0, _RPT)], deg1.at[pl.ds(r0, _RPT)])


def _make_seg(with_deg):
    mesh = plsc.VectorSubcoreMesh(core_axis_name="c", subcore_axis_name="s")
    out_type = [jax.ShapeDtypeStruct((_NPAD, _D), jnp.float32)] * 2
    scratch = [
        pltpu.VMEM((_EW,), jnp.int32),
        pltpu.VMEM((_K,), jnp.int32),
        pltpu.VMEM((_K,), jnp.int32),
        pltpu.VMEM((_K,), jnp.int32),
        pltpu.VMEM((_K, _D), jnp.float32),
        pltpu.VMEM((_K, _D), jnp.float32),
        pltpu.VMEM((_K, _D), jnp.float32),
        pltpu.VMEM((_FB, _D), jnp.float32),
        pltpu.VMEM_SHARED((_NPAD, _D), jnp.float32),
    ]
    if with_deg:
        out_type += [jax.ShapeDtypeStruct((_NPAD,), jnp.float32)] * 2
        scratch += [
            pltpu.VMEM((_K,), jnp.float32),
            pltpu.VMEM((_RPT,), jnp.float32),
            pltpu.VMEM_SHARED((_NPAD,), jnp.float32),
        ]
    scratch += [pltpu.SemaphoreType.DMA] * 6
    return pl.kernel(
        functools.partial(_seg_body, with_deg),
        out_type=tuple(out_type),
        mesh=mesh,
        scratch_types=scratch,
    )


_seg_with_deg = _make_seg(True)
_seg_no_deg = _make_seg(False)


def _enc_body(x, w, b, ei, o, so, do_):
    o[...] = jnp.maximum(
        jnp.dot(x[...], w[...], preferred_element_type=jnp.float32) + b[...], 0.0)
    so[...] = ei[0]
    do_[...] = ei[1]


def _layer_body(p0, p1, d0, d1, w, b, o):
    rdeg = 1.0 / jnp.maximum(d0[...] + d1[...], 1.0)
    agg = (p0[...] + p1[...]) * rdeg
    hh = jnp.concatenate(
        [jnp.dot(agg, w[0], preferred_element_type=jnp.float32),
         jnp.dot(agg, w[1], preferred_element_type=jnp.float32)], axis=-1)
    o[...] = jnp.maximum(hh + b[...], 0.0)


def _final_body(p0, p1, d0, d1, w, b, wd, bd, out_o, h_o):
    rdeg = 1.0 / jnp.maximum(d0[...] + d1[...], 1.0)
    agg = (p0[...] + p1[...]) * rdeg
    hh = jnp.concatenate(
        [jnp.dot(agg, w[0], preferred_element_type=jnp.float32),
         jnp.dot(agg, w[1], preferred_element_type=jnp.float32)], axis=-1)
    hh = jnp.maximum(hh + b[...], 0.0)
    h_o[...] = hh
    out_o[...] = jnp.dot(hh, wd[...], preferred_element_type=jnp.float32) + bd[...]


_row_spec = pl.BlockSpec((_BR, _D), lambda i: (i, 0))
_deg_spec = pl.BlockSpec((_BR, 1), lambda i: (i, 0))
_w_spec = pl.BlockSpec((_D, _D), lambda i: (0, 0))
_wh_spec = pl.BlockSpec((2, _D, _D // 2), lambda i: (0, 0, 0))
_b_spec = pl.BlockSpec((1, _D), lambda i: (0, 0))

_enc_call = pl.pallas_call(
    _enc_body,
    out_shape=(jax.ShapeDtypeStruct((_N, _D), jnp.float32),
               jax.ShapeDtypeStruct((_E,), jnp.int32),
               jax.ShapeDtypeStruct((_E,), jnp.int32)),
)

_layer_call = pl.pallas_call(
    _layer_body,
    grid=(_G,),
    in_specs=[_row_spec, _row_spec, _deg_spec, _deg_spec, _wh_spec, _b_spec],
    out_specs=_row_spec,
    out_shape=jax.ShapeDtypeStruct((_N, _D), jnp.float32),
)

_final_call = pl.pallas_call(
    _final_body,
    grid=(_G,),
    in_specs=[_row_spec, _row_spec, _deg_spec, _deg_spec, _wh_spec, _b_spec,
              _w_spec, _b_spec],
    out_specs=(_row_spec, _row_spec),
    out_shape=(jax.ShapeDtypeStruct((_N, _D), jnp.float32),
               jax.ShapeDtypeStruct((_N, _D), jnp.float32)),
)


def kernel(x, edge_index, enc_W, enc_b, conv_Ws, conv_bs, dec_W, dec_b):


    h0, src1, dst1 = _enc_call(x, enc_W, enc_b.reshape(1, _D), edge_index)

    pa, pb, dg0, dg1 = _seg_with_deg(h0, src1, dst1)
    d0 = dg0.reshape(_NPAD, 1)
    d1 = dg1.reshape(_NPAD, 1)

    h1 = _layer_call(pa, pb, d0, d1, conv_Ws[0], conv_bs[0].reshape(1, _D))

    qa, qb = _seg_no_deg(h1, src1, dst1)

    out, h2 = _final_call(qa, qb, d0, d1, conv_Ws[1], conv_bs[1].reshape(1, _D),
                          dec_W, dec_b.reshape(1, _D))
    return (out, h2)
